# baseline (speedup 1.0000x reference)
import functools

import jax
import jax.numpy as jnp
from jax.experimental import pallas as pl
from jax.experimental.pallas import tpu as pltpu

_BATCH = 256
_NSEM = 8


def _gather(idx, emb):
    def body(idx_ref, in_ref, out_ref, sems):
        def start(i, s):
            pltpu.make_async_copy(
                in_ref.at[idx_ref[i]], out_ref.at[i], sems.at[s]).start()

        def wait(i, s):
            pltpu.make_async_copy(
                in_ref.at[idx_ref[i]], out_ref.at[i], sems.at[s]).wait()

        for i in range(_NSEM):
            start(i, i)

        def loop(i, carry):
            s = jax.lax.rem(i, _NSEM)
            wait(i, s)

            @pl.when(i + _NSEM < _BATCH)
            def _():
                start(i + _NSEM, s)

            return carry

        jax.lax.fori_loop(0, _BATCH, loop, 0)

    grid_spec = pltpu.PrefetchScalarGridSpec(
        num_scalar_prefetch=1,
        grid=(),
        in_specs=[pl.BlockSpec(memory_space=pltpu.MemorySpace.HBM)],
        out_specs=pl.BlockSpec(memory_space=pltpu.MemorySpace.HBM),
        scratch_shapes=[pltpu.SemaphoreType.DMA((_NSEM,))],
    )
    return pl.pallas_call(
        body,
        grid_spec=grid_spec,
        out_shape=jax.ShapeDtypeStruct((_BATCH, 77, 4096), jnp.float32),
    )(idx, emb)


def kernel(prompt_idx, embeddings):
    return _gather(prompt_idx.astype(jnp.int32), embeddings)


# speedup vs baseline: 7.2737x; 7.2737x over previous
import functools

import jax
import jax.numpy as jnp
from jax import lax
from jax.experimental import pallas as pl
from jax.experimental.pallas import tpu as pltpu
from jax.experimental.pallas import tpu_sc as plsc

_BATCH = 256
_SEQ = 77
_DIM = 4096
_NC = 2
_PPC = _BATCH // _NC
_NB = 4


def _sc_gather(idx, emb):
    mesh = plsc.ScalarSubcoreMesh(axis_name="c", num_cores=_NC)

    @functools.partial(
        pl.kernel,
        out_type=jax.ShapeDtypeStruct((_BATCH, _SEQ, _DIM), jnp.float32),
        mesh=mesh,
        scratch_types=[
            pltpu.SMEM((_PPC,), jnp.int32),
            pltpu.VMEM_SHARED((_NB, _SEQ, _DIM), jnp.float32),
            pltpu.SemaphoreType.DMA((_NB,)),
            pltpu.SemaphoreType.DMA((_NB,)),
        ],
    )
    def k(idx_hbm, emb_hbm, out_hbm, pids, buf, gsem, wsem):
        c = lax.axis_index("c")
        base = c * _PPC

        pltpu.sync_copy(idx_hbm.at[pl.ds(base, _PPC)], pids)

        def start_g(i, s):
            pltpu.async_copy(emb_hbm.at[pids[i]], buf.at[s], gsem.at[s])

        def wait_g(s):
            pltpu.make_async_copy(emb_hbm.at[0], buf.at[s],
                                  gsem.at[s]).wait()

        def start_w(i, s):
            pltpu.async_copy(buf.at[s], out_hbm.at[base + i], wsem.at[s])

        def wait_w(s):
            pltpu.make_async_copy(buf.at[0], out_hbm.at[0],
                                  wsem.at[s]).wait()

        for s in range(_NB):
            start_g(s, s)

        def loop(i, carry):
            s = lax.rem(i, _NB)
            wait_g(s)
            start_w(i, s)

            @pl.when(i + _NB < _PPC)
            def _():
                wait_w(s)
                start_g(i + _NB, s)

            return carry

        lax.fori_loop(0, _PPC, loop, 0)
        for s in range(_NB):
            wait_w(s)

    return k(idx, emb)


def kernel(prompt_idx, embeddings):
    return _sc_gather(prompt_idx.astype(jnp.int32), embeddings)


# speedup vs baseline: 7.4975x; 1.0308x over previous
import jax
import jax.numpy as jnp
from jax.experimental import pallas as pl
from jax.experimental.pallas import tpu as pltpu

_BATCH = 256
_SEQ = 77
_DIM = 4096
_NS = 8
_NT = _BATCH // _NS


def _gather(idx, emb):
    def body(idx_ref, in_ref, out_ref, bufs, gsem, wsem):
        def start_g(i, s):
            pltpu.make_async_copy(
                in_ref.at[idx_ref[i]], bufs.at[s], gsem.at[s]).start()

        def wait_g(s):
            pltpu.make_async_copy(
                in_ref.at[0], bufs.at[s], gsem.at[s]).wait()

        def start_w(i, s):
            pltpu.make_async_copy(
                bufs.at[s], out_ref.at[i], wsem.at[s]).start()

        def wait_w(s):
            pltpu.make_async_copy(
                bufs.at[0], out_ref.at[0], wsem.at[s]).wait()

        for s in range(_NS):
            start_g(s, s)

        def loop(t, carry):
            for s in range(_NS):
                wait_g(s)
                start_w(t * _NS + s, s)
            for s in range(_NS):
                wait_w(s)

                @pl.when(t + 1 < _NT)
                def _():
                    start_g((t + 1) * _NS + s, s)

            return carry

        jax.lax.fori_loop(0, _NT, loop, 0)

    grid_spec = pltpu.PrefetchScalarGridSpec(
        num_scalar_prefetch=1,
        grid=(),
        in_specs=[pl.BlockSpec(memory_space=pltpu.MemorySpace.HBM)],
        out_specs=pl.BlockSpec(memory_space=pltpu.MemorySpace.HBM),
        scratch_shapes=[
            pltpu.VMEM((_NS, _SEQ, _DIM), jnp.float32),
            pltpu.SemaphoreType.DMA((_NS,)),
            pltpu.SemaphoreType.DMA((_NS,)),
        ],
    )
    return pl.pallas_call(
        body,
        grid_spec=grid_spec,
        out_shape=jax.ShapeDtypeStruct((_BATCH, _SEQ, _DIM), jnp.float32),
    )(idx, emb)


def kernel(prompt_idx, embeddings):
    return _gather(prompt_idx.astype(jnp.int32), embeddings)
